# baseline (speedup 1.0000x reference)
import jax
import jax.numpy as jnp
from jax import lax
from jax.experimental import pallas as pl
from jax.experimental.pallas import tpu as pltpu
from jax.experimental.pallas import tpu_sc as plsc

BATCH = 16384
GMF_DIM = 16
MLP_DIM = 32
H1 = 64
H2 = 32

_NC = 2
_NS = 16
_NW = _NC * _NS
_BPW = BATCH // _NW


def _gather_body(user_hbm, item_hbm, gu_w, gi_w, mu_w, mi_w,
                 gu_out, gi_out, mu_out, mi_out,
                 uidx, iidx, gu_buf, gi_buf, mu_buf, mi_buf,
                 s0, s1, s2, s3):
    wid = lax.axis_index("s") * _NC + lax.axis_index("c")
    base = wid * _BPW
    pltpu.sync_copy(user_hbm.at[pl.ds(base, _BPW)], uidx)
    pltpu.sync_copy(item_hbm.at[pl.ds(base, _BPW)], iidx)
    c0 = pltpu.async_copy(gu_w.at[uidx], gu_buf, s0)
    c1 = pltpu.async_copy(gi_w.at[iidx], gi_buf, s1)
    c2 = pltpu.async_copy(mu_w.at[uidx], mu_buf, s2)
    c3 = pltpu.async_copy(mi_w.at[iidx], mi_buf, s3)
    c0.wait()
    c1.wait()
    c2.wait()
    c3.wait()
    pltpu.sync_copy(gu_buf, gu_out.at[pl.ds(base, _BPW)])
    pltpu.sync_copy(gi_buf, gi_out.at[pl.ds(base, _BPW)])
    pltpu.sync_copy(mu_buf, mu_out.at[pl.ds(base, _BPW)])
    pltpu.sync_copy(mi_buf, mi_out.at[pl.ds(base, _BPW)])


def _make_gather():
    mesh = plsc.VectorSubcoreMesh(core_axis_name="c", subcore_axis_name="s")
    return pl.kernel(
        _gather_body,
        mesh=mesh,
        compiler_params=pltpu.CompilerParams(use_tc_tiling_on_sc=False),
        out_type=[
            jax.ShapeDtypeStruct((BATCH, GMF_DIM), jnp.float32),
            jax.ShapeDtypeStruct((BATCH, GMF_DIM), jnp.float32),
            jax.ShapeDtypeStruct((BATCH, MLP_DIM), jnp.float32),
            jax.ShapeDtypeStruct((BATCH, MLP_DIM), jnp.float32),
        ],
        scratch_types=[
            pltpu.VMEM((_BPW,), jnp.int32),
            pltpu.VMEM((_BPW,), jnp.int32),
            pltpu.VMEM((_BPW, GMF_DIM), jnp.float32),
            pltpu.VMEM((_BPW, GMF_DIM), jnp.float32),
            pltpu.VMEM((_BPW, MLP_DIM), jnp.float32),
            pltpu.VMEM((_BPW, MLP_DIM), jnp.float32),
            pltpu.SemaphoreType.DMA,
            pltpu.SemaphoreType.DMA,
            pltpu.SemaphoreType.DMA,
            pltpu.SemaphoreType.DMA,
        ],
    )


def _mlp_body(gu, gi, mu, mi, w1a, w1b, b1, w2, b2, wog, woh, bo, out):
    h1 = jnp.dot(mu[...], w1a[...], preferred_element_type=jnp.float32)
    h1 = h1 + jnp.dot(mi[...], w1b[...], preferred_element_type=jnp.float32)
    h1 = jnp.maximum(h1 + b1[...], 0.0)
    h2 = jnp.dot(h1, w2[...], preferred_element_type=jnp.float32)
    h2 = jnp.maximum(h2 + b2[...], 0.0)
    gmf = gu[...] * gi[...]
    logit = (jnp.sum(gmf * wog[...], axis=1, keepdims=True)
             + jnp.sum(h2 * woh[...], axis=1, keepdims=True)
             + bo[...])
    out[...] = 1.0 / (1.0 + jnp.exp(-logit))


_BLK = 2048


def _run_mlp(gu, gi, mu, mi, w1a, w1b, b1, w2, b2, wog, woh, bo):
    n_blocks = BATCH // _BLK
    full = lambda shape: pl.BlockSpec(shape, lambda i: (0, 0))
    return pl.pallas_call(
        _mlp_body,
        grid=(n_blocks,),
        in_specs=[
            pl.BlockSpec((_BLK, GMF_DIM), lambda i: (i, 0)),
            pl.BlockSpec((_BLK, GMF_DIM), lambda i: (i, 0)),
            pl.BlockSpec((_BLK, MLP_DIM), lambda i: (i, 0)),
            pl.BlockSpec((_BLK, MLP_DIM), lambda i: (i, 0)),
            full((MLP_DIM, H1)),
            full((MLP_DIM, H1)),
            full((1, H1)),
            full((H1, H2)),
            full((1, H2)),
            full((1, GMF_DIM)),
            full((1, H2)),
            full((1, 1)),
        ],
        out_specs=pl.BlockSpec((_BLK, 1), lambda i: (i, 0)),
        out_shape=jax.ShapeDtypeStruct((BATCH, 1), jnp.float32),
    )(gu, gi, mu, mi, w1a, w1b, b1, w2, b2, wog, woh, bo)


def kernel(user, item, gmf_user_w, gmf_item_w, mlp_user_w, mlp_item_w,
           W1, b1, W2, b2, Wo, bo):
    user = user.astype(jnp.int32)
    item = item.astype(jnp.int32)
    gu, gi, mu, mi = _make_gather()(
        user, item, gmf_user_w, gmf_item_w, mlp_user_w, mlp_item_w)
    w1a = W1[:MLP_DIM]
    w1b = W1[MLP_DIM:]
    wog = Wo[:GMF_DIM, 0].reshape(1, GMF_DIM)
    woh = Wo[GMF_DIM:, 0].reshape(1, H2)
    out = _run_mlp(gu, gi, mu, mi, w1a, w1b, b1.reshape(1, H1), W2,
                   b2.reshape(1, H2), wog, woh, bo.reshape(1, 1))
    return out.reshape(BATCH)
